# baseline (speedup 1.0000x reference)
import functools
import itertools

import jax
import jax.numpy as jnp
from jax import lax
from jax.experimental import pallas as pl
from jax.experimental.pallas import tpu as pltpu
from jax.experimental.pallas import tpu_sc as plsc

B = 2
S = 96
N = B * S * S * S
G = S + 2
GE = S + 3
ZP = 104
RN = B * G * G * G
NW = 32

NPLANES = B * G
HLINES = 49
NTASKS = NPLANES * 2
TPW = 13
IN_WORDS = (HLINES + 1) * ZP
HP_ROWS = HLINES * G
ZSTARTS7 = 7

PER_W = N // NW
PLANE = S * S
CHUNK = PLANE // 4
NCHUNK = PER_W // CHUNK
NB = CHUNK // 128
GROUPS = CHUNK // 16

_mesh = plsc.VectorSubcoreMesh(core_axis_name="c", subcore_axis_name="s")
_cp = pltpu.CompilerParams(needs_layout_passes=False, use_tc_tiling_on_sc=False)


@functools.partial(
    pl.kernel,
    out_type=jax.ShapeDtypeStruct((RN * 8,), jnp.float32),
    mesh=_mesh,
    compiler_params=_cp,
    scratch_types=[
        pltpu.VMEM((2, IN_WORDS), jnp.float32),
        pltpu.VMEM((2, IN_WORDS), jnp.float32),
        pltpu.VMEM((HP_ROWS * 8,), jnp.float32),
        pltpu.VMEM((HP_ROWS * 8,), jnp.float32),
        pltpu.SemaphoreType.DMA,
        pltpu.SemaphoreType.DMA,
        pltpu.SemaphoreType.DMA,
        pltpu.SemaphoreType.DMA,
    ],
)
def _sc_build(ext_hbm, p8_hbm,
              in_a, in_b, out_a, out_b, si_a, si_b, so_a, so_b):
    wid = lax.axis_index("s") * 2 + lax.axis_index("c")
    lane = lax.iota(jnp.int32, 16)

    def task_id(i):
        return jnp.minimum(wid + i * NW, NTASKS - 1)

    def in_issue(i, in_v, sem):
        hp = task_id(i)
        q = hp // 2
        half = hp - q * 2
        b = (q >= G).astype(jnp.int32)
        y = q - b * G
        x0 = half * HLINES
        src0 = ((b * GE + y) * GE + x0) * ZP
        pltpu.async_copy(ext_hbm.at[pl.ds(src0, IN_WORDS)], in_v.at[0], sem)
        pltpu.async_copy(ext_hbm.at[pl.ds(src0 + GE * ZP, IN_WORDS)],
                         in_v.at[1], sem)

    def in_wait(in_v, sem):
        pltpu.make_async_copy(ext_hbm.at[pl.ds(0, IN_WORDS)], in_v.at[0], sem).wait()
        pltpu.make_async_copy(ext_hbm.at[pl.ds(0, IN_WORDS)], in_v.at[1], sem).wait()

    def out_wait(out_v, sem):
        pltpu.make_async_copy(
            p8_hbm.at[pl.ds(0, HP_ROWS * 8)], out_v, sem).wait()

    def compute_issue(i, in_v, out_v, sem):
        hp = task_id(i)

        def line_body(x, carry1):
            def zgroup(zg, carry2):
                z0 = jnp.minimum(zg * 16, G - 16)
                z0v = z0 + lane
                rloc8 = (x * G + z0v) * 8
                rglob = hp * HP_ROWS + x * G + z0v
                c = 0
                for cy, cx, cz in itertools.product((0, 1), repeat=3):
                    v = in_v[cy, pl.ds((x + cx) * ZP + z0 + cz, 16)]
                    plsc.store_scatter(out_v, [rloc8 + ((rglob + c) & 7)], v)
                    c += 1
                return carry2

            return lax.fori_loop(0, ZSTARTS7, zgroup, carry1)

        lax.fori_loop(0, HLINES, line_body, None)
        pltpu.async_copy(out_v, p8_hbm.at[pl.ds(hp * (HP_ROWS * 8), HP_ROWS * 8)], sem)

    in_issue(0, in_a, si_a)
    in_issue(1, in_b, si_b)

    def pipe(it, carry):
        e = 2 * it
        o = e + 1
        in_wait(in_a, si_a)

        @pl.when(e >= 2)
        def _():
            out_wait(out_a, so_a)

        compute_issue(e, in_a, out_a, so_a)
        in_issue(e + 2, in_a, si_a)
        in_wait(in_b, si_b)

        @pl.when(o >= 2)
        def _():
            out_wait(out_b, so_b)

        compute_issue(o, in_b, out_b, so_b)

        @pl.when(o + 2 < TPW)
        def _():
            in_issue(o + 2, in_b, si_b)

        return carry

    lax.fori_loop(0, TPW // 2, pipe, None)

    in_wait(in_a, si_a)
    out_wait(out_a, so_a)
    compute_issue(TPW - 1, in_a, out_a, so_a)
    out_wait(out_a, so_a)
    out_wait(out_b, so_b)


@functools.partial(
    pl.kernel,
    out_type=jax.ShapeDtypeStruct((N,), jnp.float32),
    mesh=_mesh,
    compiler_params=_cp,
    scratch_types=[
        pltpu.VMEM((3, CHUNK), jnp.float32),
        pltpu.VMEM((3, CHUNK), jnp.float32),
        pltpu.VMEM((NB, 128), jnp.int32),
        pltpu.VMEM((NB, 128), jnp.int32),
        pltpu.VMEM((CHUNK, 8), jnp.float32),
        pltpu.VMEM((CHUNK, 8), jnp.float32),
        pltpu.VMEM((6, CHUNK), jnp.float32),
        pltpu.VMEM((6, CHUNK), jnp.float32),
        pltpu.VMEM((CHUNK,), jnp.float32),
        pltpu.SemaphoreType.DMA,
        pltpu.SemaphoreType.DMA,
        pltpu.SemaphoreType.DMA,
        pltpu.SemaphoreType.DMA,
    ],
)
def _sc_interp(p8_hbm, def_hbm, out_hbm, def_a, def_b, idx_a, idx_b,
               rows_a, rows_b, w_a, w_b, out_v, sem_a, sem_b, sem_da, sem_db):
    wid = lax.axis_index("s") * 2 + lax.axis_index("c")
    base = wid * PER_W
    lane = lax.iota(jnp.int32, 16)

    def floor_i32(v):
        return (v + 1024.0).astype(jnp.int32) - 1024

    def def_issue(ci, def_v, sem):
        off = base + ci * CHUNK
        plane = off // PLANE
        pr0 = off - plane * PLANE
        dbase = plane * (3 * PLANE) + pr0
        for ch in range(3):
            pltpu.async_copy(def_hbm.at[pl.ds(dbase + ch * PLANE, CHUNK)],
                             def_v.at[ch], sem)

    def def_wait(def_v, sem):
        for ch in range(3):
            pltpu.make_async_copy(def_hbm.at[pl.ds(0, CHUNK)],
                                  def_v.at[ch], sem).wait()

    def pass1(ci, def_v, idx_v, w_v):
        off = base + ci * CHUNK
        plane = off // PLANE
        pr0 = off - plane * PLANE
        b = (plane >= S).astype(jnp.int32)
        i = plane - b * S
        j0 = pr0 // S
        i_f = i.astype(jnp.float32)

        def body(g, carry1):
            loc = g * 16
            j = j0 + g // 6
            k = (g % 6) * 16 + lane
            dxv = def_v[0, pl.ds(loc, 16)]
            dyv = def_v[1, pl.ds(loc, 16)]
            dzv = def_v[2, pl.ds(loc, 16)]
            xf = (dxv + j.astype(jnp.float32)) + 1.0
            yf = (dyv + i_f) + 1.0
            zf = (dzv + k.astype(jnp.float32)) + 1.0
            xi = floor_i32(xf)
            yi = floor_i32(yf)
            zi = floor_i32(zf)
            ax = jnp.minimum(jnp.maximum(xi, 0), G - 1)
            ay = jnp.minimum(jnp.maximum(yi, 0), G - 1)
            az = jnp.minimum(jnp.maximum(zi, 0), G - 1)
            gx0 = (xi + 1).astype(jnp.float32) - xf
            gy0 = (yi + 1).astype(jnp.float32) - yf
            gz0 = (zi + 1).astype(jnp.float32) - zf
            zero = jnp.zeros((16,), jnp.float32)
            gx1 = jnp.where(xi >= 0, 1.0 - gx0, zero)
            gy1 = jnp.where(yi >= 0, 1.0 - gy0, zero)
            gz1 = jnp.where(zi >= 0, 1.0 - gz0, zero)
            r = ((b * G + ay) * G + ax) * G + az
            idx_v[g // 8, pl.ds((g % 8) * 16, 16)] = r
            w_v[0, pl.ds(loc, 16)] = gy0
            w_v[1, pl.ds(loc, 16)] = gy1
            w_v[2, pl.ds(loc, 16)] = gx0
            w_v[3, pl.ds(loc, 16)] = gx1
            w_v[4, pl.ds(loc, 16)] = gz0
            w_v[5, pl.ds(loc, 16)] = gz1
            return carry1

        lax.fori_loop(0, GROUPS, body, None)

    def gather_issue(idx_v, rows_v, sem):
        for t in range(NB):
            pltpu.async_copy(
                p8_hbm.at[idx_v.at[t]],
                rows_v.at[pl.ds(t * 128, 128)],
                sem,
            )

    def gather_wait(rows_v, sem):
        pltpu.make_async_copy(p8_hbm.at[pl.ds(0, CHUNK)], rows_v, sem).wait()

    def pass2(ci, rows_v, idx_v, w_v):
        def body(g, carry2):
            loc = g * 16
            l = loc + lane
            gy0 = w_v[0, pl.ds(loc, 16)]
            gy1 = w_v[1, pl.ds(loc, 16)]
            gx0 = w_v[2, pl.ds(loc, 16)]
            gx1 = w_v[3, pl.ds(loc, 16)]
            gz0 = w_v[4, pl.ds(loc, 16)]
            gz1 = w_v[5, pl.ds(loc, 16)]
            rv = plsc.load_gather(idx_v, [jnp.full((16,), g // 8, jnp.int32), (g % 8) * 16 + lane])
            vs = []
            for c in range(8):
                vs.append(plsc.load_gather(rows_v, [l, (rv + c) & 7]))
            t00 = gz0 * vs[0] + gz1 * vs[1]
            t01 = gz0 * vs[2] + gz1 * vs[3]
            t10 = gz0 * vs[4] + gz1 * vs[5]
            t11 = gz0 * vs[6] + gz1 * vs[7]
            u0 = gx0 * t00 + gx1 * t01
            u1 = gx0 * t10 + gx1 * t11
            out_v[pl.ds(loc, 16)] = gy0 * u0 + gy1 * u1
            return carry2

        lax.fori_loop(0, GROUPS, body, None)
        off = base + ci * CHUNK
        pltpu.sync_copy(out_v, out_hbm.at[pl.ds(off, CHUNK)])

    def_issue(0, def_a, sem_da)
    def_wait(def_a, sem_da)
    pass1(0, def_a, idx_a, w_a)
    gather_issue(idx_a, rows_a, sem_a)
    def_issue(1, def_b, sem_db)

    def pipe(it, carry):
        e = 2 * it
        o = e + 1
        def_wait(def_b, sem_db)
        pass1(o, def_b, idx_b, w_b)
        gather_wait(rows_a, sem_a)
        gather_issue(idx_b, rows_b, sem_b)
        def_issue(e + 2, def_a, sem_da)
        pass2(e, rows_a, idx_a, w_a)
        def_wait(def_a, sem_da)
        pass1(e + 2, def_a, idx_a, w_a)
        gather_wait(rows_b, sem_b)
        gather_issue(idx_a, rows_a, sem_a)
        def_issue(o + 2, def_b, sem_db)
        pass2(o, rows_b, idx_b, w_b)
        return carry

    lax.fori_loop(0, NCHUNK // 2 - 1, pipe, None)

    o = NCHUNK - 1
    def_wait(def_b, sem_db)
    pass1(o, def_b, idx_b, w_b)
    gather_wait(rows_a, sem_a)
    gather_issue(idx_b, rows_b, sem_b)
    pass2(NCHUNK - 2, rows_a, idx_a, w_a)
    gather_wait(rows_b, sem_b)
    pass2(o, rows_b, idx_b, w_b)


def kernel(moving_image, deformation_matrix):
    ext = jnp.pad(moving_image, ((0, 0), (1, 2), (1, 2), (1, ZP - S - 1)))
    p8 = _sc_build(ext.reshape(-1)).reshape(RN, 8)
    def_flat = jnp.transpose(deformation_matrix, (0, 1, 4, 2, 3)).reshape(-1)
    out = _sc_interp(p8, def_flat)
    return out.reshape(B, S, S, S)


# speedup vs baseline: 1.0290x; 1.0290x over previous
import functools
import itertools

import jax
import jax.numpy as jnp
from jax import lax
from jax.experimental import pallas as pl
from jax.experimental.pallas import tpu as pltpu
from jax.experimental.pallas import tpu_sc as plsc

B = 2
S = 96
N = B * S * S * S
G = S + 2
GE = S + 3
ZP = 104
RN = B * G * G * G
NW = 32

NPLANES = B * G
HLINES = 49
NTASKS = NPLANES * 2
TPW = 13
IN_WORDS = (HLINES + 1) * ZP
HP_ROWS = HLINES * G
ZSTARTS7 = 7

PER_W = N // NW
PLANE = S * S
CHUNK = PLANE // 4
NCHUNK = PER_W // CHUNK
NB = CHUNK // 128
GROUPS = CHUNK // 16

_mesh = plsc.VectorSubcoreMesh(core_axis_name="c", subcore_axis_name="s")
_cp = pltpu.CompilerParams(needs_layout_passes=False, use_tc_tiling_on_sc=False)


@functools.partial(
    pl.kernel,
    out_type=jax.ShapeDtypeStruct((RN * 8,), jnp.float32),
    mesh=_mesh,
    compiler_params=_cp,
    scratch_types=[
        pltpu.VMEM((2, IN_WORDS), jnp.float32),
        pltpu.VMEM((2, IN_WORDS), jnp.float32),
        pltpu.VMEM((HP_ROWS * 8,), jnp.float32),
        pltpu.VMEM((HP_ROWS * 8,), jnp.float32),
        pltpu.SemaphoreType.DMA,
        pltpu.SemaphoreType.DMA,
        pltpu.SemaphoreType.DMA,
        pltpu.SemaphoreType.DMA,
    ],
)
def _sc_build(ext_hbm, p8_hbm,
              in_a, in_b, out_a, out_b, si_a, si_b, so_a, so_b):
    wid = lax.axis_index("s") * 2 + lax.axis_index("c")
    lane = lax.iota(jnp.int32, 16)

    def task_id(i):
        return jnp.minimum(wid + i * NW, NTASKS - 1)

    def in_issue(i, in_v, sem):
        hp = task_id(i)
        q = hp // 2
        half = hp - q * 2
        b = (q >= G).astype(jnp.int32)
        y = q - b * G
        x0 = half * HLINES
        src0 = ((b * GE + y) * GE + x0) * ZP
        pltpu.async_copy(ext_hbm.at[pl.ds(src0, IN_WORDS)], in_v.at[0], sem)
        pltpu.async_copy(ext_hbm.at[pl.ds(src0 + GE * ZP, IN_WORDS)],
                         in_v.at[1], sem)

    def in_wait(in_v, sem):
        pltpu.make_async_copy(ext_hbm.at[pl.ds(0, IN_WORDS)], in_v.at[0], sem).wait()
        pltpu.make_async_copy(ext_hbm.at[pl.ds(0, IN_WORDS)], in_v.at[1], sem).wait()

    def out_wait(out_v, sem):
        pltpu.make_async_copy(
            p8_hbm.at[pl.ds(0, HP_ROWS * 8)], out_v, sem).wait()

    _cyv = lax.iota(jnp.int32, 16)
    CYV = (_cyv >> 2) & 1
    LREST = ((_cyv >> 1) & 1) * ZP + (_cyv & 1) + (_cyv >> 3)

    def compute_issue(i, in_v, out_v, sem):
        hp = task_id(i)

        def line_body(x, carry1):
            xb = x * ZP
            ob = x * (G * 8)

            def zgroup(zi7, carry2):
                for u in range(7):
                    z0 = zi7 * 14 + u * 2
                    v = plsc.load_gather(in_v, [CYV, LREST + (xb + z0)])
                    out_v[pl.ds(ob + z0 * 8, 16)] = v
                return carry2

            return lax.fori_loop(0, 7, zgroup, carry1)

        lax.fori_loop(0, HLINES, line_body, None)
        pltpu.async_copy(out_v, p8_hbm.at[pl.ds(hp * (HP_ROWS * 8), HP_ROWS * 8)], sem)

    in_issue(0, in_a, si_a)
    in_issue(1, in_b, si_b)

    def pipe(it, carry):
        e = 2 * it
        o = e + 1
        in_wait(in_a, si_a)

        @pl.when(e >= 2)
        def _():
            out_wait(out_a, so_a)

        compute_issue(e, in_a, out_a, so_a)
        in_issue(e + 2, in_a, si_a)
        in_wait(in_b, si_b)

        @pl.when(o >= 2)
        def _():
            out_wait(out_b, so_b)

        compute_issue(o, in_b, out_b, so_b)

        @pl.when(o + 2 < TPW)
        def _():
            in_issue(o + 2, in_b, si_b)

        return carry

    lax.fori_loop(0, TPW // 2, pipe, None)

    in_wait(in_a, si_a)
    out_wait(out_a, so_a)
    compute_issue(TPW - 1, in_a, out_a, so_a)
    out_wait(out_a, so_a)
    out_wait(out_b, so_b)


@functools.partial(
    pl.kernel,
    out_type=jax.ShapeDtypeStruct((N,), jnp.float32),
    mesh=_mesh,
    compiler_params=_cp,
    scratch_types=[
        pltpu.VMEM((3, CHUNK), jnp.float32),
        pltpu.VMEM((3, CHUNK), jnp.float32),
        pltpu.VMEM((NB, 128), jnp.int32),
        pltpu.VMEM((NB, 128), jnp.int32),
        pltpu.VMEM((CHUNK, 8), jnp.float32),
        pltpu.VMEM((CHUNK, 8), jnp.float32),
        pltpu.VMEM((6, CHUNK), jnp.float32),
        pltpu.VMEM((6, CHUNK), jnp.float32),
        pltpu.VMEM((CHUNK,), jnp.float32),
        pltpu.SemaphoreType.DMA,
        pltpu.SemaphoreType.DMA,
        pltpu.SemaphoreType.DMA,
        pltpu.SemaphoreType.DMA,
    ],
)
def _sc_interp(p8_hbm, def_hbm, out_hbm, def_a, def_b, idx_a, idx_b,
               rows_a, rows_b, w_a, w_b, out_v, sem_a, sem_b, sem_da, sem_db):
    wid = lax.axis_index("s") * 2 + lax.axis_index("c")
    base = wid * PER_W
    lane = lax.iota(jnp.int32, 16)

    def floor_i32(v):
        return (v + 1024.0).astype(jnp.int32) - 1024

    def def_issue(ci, def_v, sem):
        off = base + ci * CHUNK
        plane = off // PLANE
        pr0 = off - plane * PLANE
        dbase = plane * (3 * PLANE) + pr0
        for ch in range(3):
            pltpu.async_copy(def_hbm.at[pl.ds(dbase + ch * PLANE, CHUNK)],
                             def_v.at[ch], sem)

    def def_wait(def_v, sem):
        for ch in range(3):
            pltpu.make_async_copy(def_hbm.at[pl.ds(0, CHUNK)],
                                  def_v.at[ch], sem).wait()

    def pass1(ci, def_v, idx_v, w_v):
        off = base + ci * CHUNK
        plane = off // PLANE
        pr0 = off - plane * PLANE
        b = (plane >= S).astype(jnp.int32)
        i = plane - b * S
        j0 = pr0 // S
        i_f = i.astype(jnp.float32)

        def body(g, carry1):
            loc = g * 16
            j = j0 + g // 6
            k = (g % 6) * 16 + lane
            dxv = def_v[0, pl.ds(loc, 16)]
            dyv = def_v[1, pl.ds(loc, 16)]
            dzv = def_v[2, pl.ds(loc, 16)]
            xf = (dxv + j.astype(jnp.float32)) + 1.0
            yf = (dyv + i_f) + 1.0
            zf = (dzv + k.astype(jnp.float32)) + 1.0
            xi = floor_i32(xf)
            yi = floor_i32(yf)
            zi = floor_i32(zf)
            ax = jnp.minimum(jnp.maximum(xi, 0), G - 1)
            ay = jnp.minimum(jnp.maximum(yi, 0), G - 1)
            az = jnp.minimum(jnp.maximum(zi, 0), G - 1)
            gx0 = (xi + 1).astype(jnp.float32) - xf
            gy0 = (yi + 1).astype(jnp.float32) - yf
            gz0 = (zi + 1).astype(jnp.float32) - zf
            zero = jnp.zeros((16,), jnp.float32)
            gx1 = jnp.where(xi >= 0, 1.0 - gx0, zero)
            gy1 = jnp.where(yi >= 0, 1.0 - gy0, zero)
            gz1 = jnp.where(zi >= 0, 1.0 - gz0, zero)
            r = ((b * G + ay) * G + ax) * G + az
            idx_v[g // 8, pl.ds((g % 8) * 16, 16)] = r
            w_v[0, pl.ds(loc, 16)] = gy0
            w_v[1, pl.ds(loc, 16)] = gy1
            w_v[2, pl.ds(loc, 16)] = gx0
            w_v[3, pl.ds(loc, 16)] = gx1
            w_v[4, pl.ds(loc, 16)] = gz0
            w_v[5, pl.ds(loc, 16)] = gz1
            return carry1

        lax.fori_loop(0, GROUPS, body, None)

    def gather_issue(idx_v, rows_v, sem):
        for t in range(NB):
            pltpu.async_copy(
                p8_hbm.at[idx_v.at[t]],
                rows_v.at[pl.ds(t * 128, 128)],
                sem,
            )

    def gather_wait(rows_v, sem):
        pltpu.make_async_copy(p8_hbm.at[pl.ds(0, CHUNK)], rows_v, sem).wait()

    def pass2(ci, rows_v, w_v):
        def body(g, carry2):
            loc = g * 16
            l = loc + lane
            gy0 = w_v[0, pl.ds(loc, 16)]
            gy1 = w_v[1, pl.ds(loc, 16)]
            gx0 = w_v[2, pl.ds(loc, 16)]
            gx1 = w_v[3, pl.ds(loc, 16)]
            gz0 = w_v[4, pl.ds(loc, 16)]
            gz1 = w_v[5, pl.ds(loc, 16)]
            vs = []
            for c in range(8):
                col = jnp.full((16,), c, jnp.int32)
                vs.append(plsc.load_gather(rows_v, [l, col]))
            t00 = gz0 * vs[0] + gz1 * vs[1]
            t01 = gz0 * vs[2] + gz1 * vs[3]
            t10 = gz0 * vs[4] + gz1 * vs[5]
            t11 = gz0 * vs[6] + gz1 * vs[7]
            u0 = gx0 * t00 + gx1 * t01
            u1 = gx0 * t10 + gx1 * t11
            out_v[pl.ds(loc, 16)] = gy0 * u0 + gy1 * u1
            return carry2

        lax.fori_loop(0, GROUPS, body, None)
        off = base + ci * CHUNK
        pltpu.sync_copy(out_v, out_hbm.at[pl.ds(off, CHUNK)])

    def_issue(0, def_a, sem_da)
    def_wait(def_a, sem_da)
    pass1(0, def_a, idx_a, w_a)
    gather_issue(idx_a, rows_a, sem_a)
    def_issue(1, def_b, sem_db)

    def pipe(it, carry):
        e = 2 * it
        o = e + 1
        def_wait(def_b, sem_db)
        pass1(o, def_b, idx_b, w_b)
        gather_wait(rows_a, sem_a)
        gather_issue(idx_b, rows_b, sem_b)
        def_issue(e + 2, def_a, sem_da)
        pass2(e, rows_a, w_a)
        def_wait(def_a, sem_da)
        pass1(e + 2, def_a, idx_a, w_a)
        gather_wait(rows_b, sem_b)
        gather_issue(idx_a, rows_a, sem_a)
        def_issue(o + 2, def_b, sem_db)
        pass2(o, rows_b, w_b)
        return carry

    lax.fori_loop(0, NCHUNK // 2 - 1, pipe, None)

    o = NCHUNK - 1
    def_wait(def_b, sem_db)
    pass1(o, def_b, idx_b, w_b)
    gather_wait(rows_a, sem_a)
    gather_issue(idx_b, rows_b, sem_b)
    pass2(NCHUNK - 2, rows_a, w_a)
    gather_wait(rows_b, sem_b)
    pass2(o, rows_b, w_b)


def kernel(moving_image, deformation_matrix):
    ext = jnp.pad(moving_image, ((0, 0), (1, 2), (1, 2), (1, ZP - S - 1)))
    p8 = _sc_build(ext.reshape(-1)).reshape(RN, 8)
    def_flat = jnp.transpose(deformation_matrix, (0, 1, 4, 2, 3)).reshape(-1)
    out = _sc_interp(p8, def_flat)
    return out.reshape(B, S, S, S)
